# Initial kernel scaffold; baseline (speedup 1.0000x reference)
import jax
import jax.numpy as jnp
from jax.experimental import pallas as pl


def kernel(x, sadj, fadj, asadj, afadj, params):
    raise NotImplementedError("write your pallas kernel here")


# speedup vs baseline: 2.9819x; 2.9819x over previous
import functools

import jax
import jax.numpy as jnp
from jax.experimental import pallas as pl

N = 10000
H2 = 64
C = 16
_ROWS = 1000


def _fusion_body(e1_ref, e2_ref, xc_ref, aW1_ref, ab1_ref, aW2_ref,
                 rW_ref, rb_ref, mW1_ref, mb1_ref, mW2_ref, mb2_ref, out_ref):
    e1 = e1_ref[...]
    e2 = e2_ref[...]
    xc = xc_ref[...]
    aW1 = aW1_ref[...]
    ab1 = ab1_ref[...]
    aW2 = aW2_ref[...]

    def att_w(e):
        t = jnp.tanh(jnp.dot(e, aW1, preferred_element_type=jnp.float32) + ab1)
        return jnp.sum(t * aW2, axis=1, keepdims=True)

    w1, w2, w3 = att_w(e1), att_w(e2), att_w(xc)
    m = jnp.maximum(jnp.maximum(w1, w2), w3)
    x1, x2, x3 = jnp.exp(w1 - m), jnp.exp(w2 - m), jnp.exp(w3 - m)
    den = x1 + x2 + x3
    emb = (x1 * e1 + x2 * e2 + x3 * xc) / den
    emb = emb + jnp.dot(emb, rW_ref[...], preferred_element_type=jnp.float32) + rb_ref[...]
    h = jnp.tanh(jnp.dot(emb, mW1_ref[...], preferred_element_type=jnp.float32) + mb1_ref[...])
    logits = jnp.dot(h, mW2_ref[...], preferred_element_type=jnp.float32) + mb2_ref[...]
    mx = jnp.max(logits, axis=1, keepdims=True)
    lse = jnp.log(jnp.sum(jnp.exp(logits - mx), axis=1, keepdims=True)) + mx
    out_ref[...] = logits - lse


def _fusion(e1, e2, xc, p):
    full = lambda shape: pl.BlockSpec(shape, lambda i: (0,) * len(shape))
    blk = lambda w: pl.BlockSpec((_ROWS, w), lambda i: (i, 0))
    return pl.pallas_call(
        _fusion_body,
        grid=(N // _ROWS,),
        in_specs=[blk(H2), blk(H2), blk(H2),
                  full((H2, 16)), full((1, 16)), full((1, 16)),
                  full((H2, H2)), full((1, H2)),
                  full((H2, 16)), full((1, 16)), full((16, C)), full((1, C))],
        out_specs=blk(C),
        out_shape=jax.ShapeDtypeStruct((N, C), jnp.float32),
    )(e1, e2, xc,
      p["att"]["W1"], p["att"]["b1"].reshape(1, 16), p["att"]["W2"].reshape(1, 16),
      p["res"]["W"], p["res"]["b"].reshape(1, H2),
      p["mlp"]["W1"], p["mlp"]["b1"].reshape(1, 16),
      p["mlp"]["W2"], p["mlp"]["b2"].reshape(1, C))


def _gal(p, x, src, dst):
    h = x @ p["W"]
    e = jax.nn.leaky_relu(h[src] @ p["a_src"] + h[dst] @ p["a_dst"], negative_slope=0.2)
    ex = jnp.exp(e)
    den = jax.ops.segment_sum(ex, dst, num_segments=N)
    num = jax.ops.segment_sum(ex[:, None] * h[src], dst, num_segments=N)
    return num / (den[:, None] + 1e-16)


def _gat(p, x, edge_index):
    src, dst = edge_index[0], edge_index[1]
    h = jnp.concatenate([_gal(ph, x, src, dst) for ph in p["heads"]], axis=1)
    h = jax.nn.elu(_gal(p["out"], h, src, dst))
    return jax.nn.softmax(h, axis=1)


def _gcn(p, x, edge_index):
    src, dst = edge_index[0], edge_index[1]
    s1 = x @ p["W1"]
    h = jax.nn.relu(jax.ops.segment_sum(s1[src], dst, num_segments=N) + p["b1"])
    s2 = h @ p["W2"]
    return jax.nn.relu(jax.ops.segment_sum(s2[src], dst, num_segments=N) + p["b2"])


def kernel(x, sadj, fadj, asadj, afadj, params):
    emb1 = _gat(params["gat1"], x, asadj)
    com1 = _gcn(params["gcn"], x, sadj)
    com2 = _gcn(params["gcn"], x, fadj)
    emb2 = _gat(params["gat2"], x, afadj)
    xcom = (com1 + com2) / 2.0
    return _fusion(emb1, emb2, xcom, params)


# speedup vs baseline: 12.9445x; 4.3411x over previous
import functools

import jax
import jax.numpy as jnp
from jax import lax
from jax.experimental import pallas as pl
from jax.experimental.pallas import tpu as pltpu
from jax.experimental.pallas import tpu_sc as plsc

N = 10000
E = 320000
F = 128
C = 16

NC = 2
NS = 16
NW = NC * NS
EPT = E // NW
B = 80
NB = EPT // B
D = 128
SUB_ROWS = 640
SUB_CHUNKS = 8

_R = 1000


def _copy_range(sid, src_at, dst_at):
    for j in range(SUB_CHUNKS):
        start = pl.multiple_of(sid * SUB_ROWS + j * B, 8)

        @pl.when(start < N)
        def _():
            pltpu.sync_copy(src_at(start), dst_at(start))


@functools.lru_cache(maxsize=None)
def _sc_pass(H):
    mesh = plsc.VectorSubcoreMesh(core_axis_name="c", subcore_axis_name="s",
                                  num_cores=NC, num_subcores=NS)
    scratch = [
        pltpu.VMEM((B,), jnp.int32),
        pltpu.VMEM((B,), jnp.int32),
        pltpu.VMEM((B, D), jnp.float32),
        pltpu.VMEM_SHARED((N, D), jnp.float32),
        pltpu.SemaphoreType.DMA,
    ]
    if H:
        scratch += [
            pltpu.VMEM((N,), jnp.float32),
            pltpu.VMEM((N,), jnp.float32),
            pltpu.VMEM((B,), jnp.float32),
        ]

    def body(rows_hbm, ssrc_hbm, sdst_hbm, adj_hbm, out_hbm,
             src_v, dst_v, rows_v, acc, sem, *wscratch):
        cid = lax.axis_index("c")
        sid = lax.axis_index("s")
        wid = cid * NS + sid

        def zrow(i, carry):
            for j in range(D // 16):
                rows_v[i, pl.ds(j * 16, 16)] = jnp.zeros((16,), jnp.float32)
            return carry
        lax.fori_loop(0, B, zrow, 0)
        _copy_range(sid, lambda s: rows_v, lambda s: acc.at[pl.ds(s, B)])

        if H:
            s_src_v, s_dst_v, wbuf = wscratch
            pltpu.sync_copy(ssrc_hbm, s_src_v)
            pltpu.sync_copy(sdst_hbm, s_dst_v)
        plsc.subcore_barrier()

        def chunk(k, carry):
            base = wid * EPT + k * B
            pltpu.sync_copy(adj_hbm.at[pl.ds(base, B)], src_v)
            pltpu.sync_copy(adj_hbm.at[pl.ds(E + base, B)], dst_v)
            pltpu.async_copy(rows_hbm.at[src_v], rows_v, sem).wait()
            if H:
                for e0 in range(0, B, 16):
                    idxs = src_v[pl.ds(e0, 16)]
                    idxd = dst_v[pl.ds(e0, 16)]
                    ev = (plsc.load_gather(s_src_v, [idxs])
                          + plsc.load_gather(s_dst_v, [idxd]))
                    ev = jnp.where(ev >= 0.0, ev, 0.2 * ev)
                    wbuf[pl.ds(e0, 16)] = jnp.exp(ev)

                def scale(e, carry):
                    w = plsc.load_gather(wbuf, [jnp.full((16,), e, jnp.int32)])
                    for j in range(5):
                        sl = pl.ds(j * 16, 16)
                        rows_v[e, sl] = rows_v[e, sl] * w
                    return carry
                lax.fori_loop(0, B, scale, 0)
            pltpu.sync_copy(rows_v, acc.at[dst_v], add=True)
            return carry
        lax.fori_loop(0, NB, chunk, 0)
        plsc.subcore_barrier()

        _copy_range(sid, lambda s: acc.at[pl.ds(s, B)],
                    lambda s: out_hbm.at[cid, pl.ds(s, B)])

    cp = pltpu.CompilerParams(needs_layout_passes=False)
    if H:
        return pl.kernel(body,
                         out_type=jax.ShapeDtypeStruct((NC, N, D), jnp.float32),
                         mesh=mesh, scratch_types=scratch, compiler_params=cp)
    def body0(rows_hbm, adj_hbm, out_hbm, *rest):
        return body(rows_hbm, None, None, adj_hbm, out_hbm, *rest)
    return pl.kernel(body0,
                     out_type=jax.ShapeDtypeStruct((NC, N, D), jnp.float32),
                     mesh=mesh, scratch_types=scratch, compiler_params=cp)


def _ones_pad(ref, r):
    lane = lax.broadcasted_iota(jnp.int32, (r, 16), 1)
    ref[:, 64:80] = (lane < 1).astype(jnp.float32)
    ref[:, 80:128] = jnp.zeros((r, 48), jnp.float32)


def _tca_body(x_ref, *refs):
    (w10, w11, w12, w13, w20, w21, w22, w23, as1_ref, ad1_ref, as2_ref, ad2_ref,
     wg_ref,
     h10, h11, h12, h13, h20, h21, h22, h23,
     ss1_ref, sd1_ref, ss2_ref, sd2_ref, g1_ref) = refs
    x = x_ref[...]
    for (ws, hrefs, ss_ref, sd_ref, as_ref, ad_ref) in [
            ((w10, w11, w12, w13), (h10, h11, h12, h13), ss1_ref, sd1_ref, as1_ref, ad1_ref),
            ((w20, w21, w22, w23), (h20, h21, h22, h23), ss2_ref, sd2_ref, as2_ref, ad2_ref)]:
        for k in range(4):
            h = jnp.dot(x, ws[k][...], preferred_element_type=jnp.float32)
            hrefs[k][:, 0:64] = h
            _ones_pad(hrefs[k], _R)
            ss_ref[:, k:k + 1] = jnp.sum(h * as_ref[k:k + 1, :], axis=1, keepdims=True)
            sd_ref[:, k:k + 1] = jnp.sum(h * ad_ref[k:k + 1, :], axis=1, keepdims=True)
    g1_ref[:, 0:64] = jnp.dot(x, wg_ref[...], preferred_element_type=jnp.float32)
    g1_ref[:, 64:128] = jnp.zeros((_R, 64), jnp.float32)


def _tca(x, p):
    full = lambda shape: pl.BlockSpec(shape, lambda i: (0,) * len(shape))
    blk = lambda w: pl.BlockSpec((_R, w), lambda i: (i, 0))
    sblk = pl.BlockSpec((_R, 4), lambda i: (i, 0))
    g1h, g2h = p["gat1"]["heads"], p["gat2"]["heads"]
    asrc1 = jnp.stack([h["a_src"] for h in g1h])
    adst1 = jnp.stack([h["a_dst"] for h in g1h])
    asrc2 = jnp.stack([h["a_src"] for h in g2h])
    adst2 = jnp.stack([h["a_dst"] for h in g2h])
    outs = [jax.ShapeDtypeStruct((N, D), jnp.float32)] * 8 + \
           [jax.ShapeDtypeStruct((N, 4), jnp.float32)] * 4 + \
           [jax.ShapeDtypeStruct((N, D), jnp.float32)]
    return pl.pallas_call(
        _tca_body,
        grid=(N // _R,),
        in_specs=[blk(F)] + [full((F, 64))] * 8 + [full((4, 64))] * 4 + [full((F, 64))],
        out_specs=[blk(D)] * 8 + [sblk] * 4 + [blk(D)],
        out_shape=outs,
    )(x, *[h["W"] for h in g1h], *[h["W"] for h in g2h],
      asrc1, adst1, asrc2, adst2, p["gcn"]["W1"])


def _norm(pref):
    acc = pref[0] + pref[1]
    return acc[:, 0:64] / (acc[:, 64:65] + 1e-16)


def _tcb_body(p10, p11, p12, p13, p20, p21, p22, p23, q1s_ref, q1f_ref,
              wo1_ref, as1_ref, ad1_ref, wo2_ref, as2_ref, ad2_ref,
              b1_ref, w2_ref,
              ho1_ref, so1s_ref, so1d_ref, ho2_ref, so2s_ref, so2d_ref,
              g2s_ref, g2f_ref):
    for (prefs, wo_ref, as_ref, ad_ref, ho_ref, sos_ref, sod_ref) in [
            ((p10, p11, p12, p13), wo1_ref, as1_ref, ad1_ref, ho1_ref, so1s_ref, so1d_ref),
            ((p20, p21, p22, p23), wo2_ref, as2_ref, ad2_ref, ho2_ref, so2s_ref, so2d_ref)]:
        hcat = jnp.concatenate([_norm(p) for p in prefs], axis=1)
        ho = jnp.dot(hcat, wo_ref[...], preferred_element_type=jnp.float32)
        ho_ref[:, 0:64] = ho
        _ones_pad(ho_ref, _R)
        sos_ref[...] = jnp.sum(ho * as_ref[...], axis=1, keepdims=True)
        sod_ref[...] = jnp.sum(ho * ad_ref[...], axis=1, keepdims=True)
    b1 = b1_ref[...]
    w2 = w2_ref[...]
    zero = jnp.zeros((_R, 64), jnp.float32)
    hs = jax.nn.relu(q1s_ref[0, :, 0:64] + q1s_ref[1, :, 0:64] + b1)
    hf = jax.nn.relu(q1f_ref[0, :, 0:64] + q1f_ref[1, :, 0:64] + b1)
    g2s_ref[:, 0:64] = jnp.dot(hs, w2, preferred_element_type=jnp.float32)
    g2s_ref[:, 64:128] = zero
    g2f_ref[:, 0:64] = jnp.dot(hf, w2, preferred_element_type=jnp.float32)
    g2f_ref[:, 64:128] = zero


def _tcb(ps, q1s, q1f, p):
    full = lambda shape: pl.BlockSpec(shape, lambda i: (0,) * len(shape))
    blk = lambda w: pl.BlockSpec((_R, w), lambda i: (i, 0))
    pblk = pl.BlockSpec((NC, _R, D), lambda i: (0, i, 0))
    sblk = pl.BlockSpec((_R, 1), lambda i: (i, 0))
    outs = [jax.ShapeDtypeStruct((N, D), jnp.float32),
            jax.ShapeDtypeStruct((N, 1), jnp.float32),
            jax.ShapeDtypeStruct((N, 1), jnp.float32)] * 2 + \
           [jax.ShapeDtypeStruct((N, D), jnp.float32)] * 2
    o1, o2 = p["gat1"]["out"], p["gat2"]["out"]
    return pl.pallas_call(
        _tcb_body,
        grid=(N // _R,),
        in_specs=[pblk] * 10 +
                 [full((256, 64)), full((1, 64)), full((1, 64))] * 2 +
                 [full((1, 64)), full((64, 64))],
        out_specs=[blk(D), sblk, sblk, blk(D), sblk, sblk, blk(D), blk(D)],
        out_shape=outs,
    )(*ps, q1s, q1f,
      o1["W"], o1["a_src"].reshape(1, 64), o1["a_dst"].reshape(1, 64),
      o2["W"], o2["a_src"].reshape(1, 64), o2["a_dst"].reshape(1, 64),
      p["gcn"]["b1"].reshape(1, 64), p["gcn"]["W2"])


def _tcc_body(po1_ref, po2_ref, q2s_ref, q2f_ref,
              b2_ref, aW1_ref, ab1_ref, aW2_ref, rW_ref, rb_ref,
              mW1_ref, mb1_ref, mW2_ref, mb2_ref, out_ref):
    def emb_of(po_ref):
        h = _norm(po_ref)
        h = jnp.where(h > 0.0, h, jnp.exp(jnp.minimum(h, 0.0)) - 1.0)
        m = jnp.max(h, axis=1, keepdims=True)
        ex = jnp.exp(h - m)
        return ex / jnp.sum(ex, axis=1, keepdims=True)

    e1 = emb_of(po1_ref)
    e2 = emb_of(po2_ref)
    b2 = b2_ref[...]
    com1 = jax.nn.relu(q2s_ref[0, :, 0:64] + q2s_ref[1, :, 0:64] + b2)
    com2 = jax.nn.relu(q2f_ref[0, :, 0:64] + q2f_ref[1, :, 0:64] + b2)
    xc = (com1 + com2) * 0.5

    aW1, ab1, aW2 = aW1_ref[...], ab1_ref[...], aW2_ref[...]

    def att_w(e):
        t = jnp.tanh(jnp.dot(e, aW1, preferred_element_type=jnp.float32) + ab1)
        return jnp.sum(t * aW2, axis=1, keepdims=True)

    w1, w2, w3 = att_w(e1), att_w(e2), att_w(xc)
    m = jnp.maximum(jnp.maximum(w1, w2), w3)
    x1, x2, x3 = jnp.exp(w1 - m), jnp.exp(w2 - m), jnp.exp(w3 - m)
    emb = (x1 * e1 + x2 * e2 + x3 * xc) / (x1 + x2 + x3)
    emb = emb + jnp.dot(emb, rW_ref[...], preferred_element_type=jnp.float32) + rb_ref[...]
    h = jnp.tanh(jnp.dot(emb, mW1_ref[...], preferred_element_type=jnp.float32) + mb1_ref[...])
    logits = jnp.dot(h, mW2_ref[...], preferred_element_type=jnp.float32) + mb2_ref[...]
    mx = jnp.max(logits, axis=1, keepdims=True)
    lse = jnp.log(jnp.sum(jnp.exp(logits - mx), axis=1, keepdims=True)) + mx
    out_ref[...] = logits - lse


def _tcc(po1, po2, q2s, q2f, p):
    full = lambda shape: pl.BlockSpec(shape, lambda i: (0,) * len(shape))
    blk = lambda w: pl.BlockSpec((_R, w), lambda i: (i, 0))
    pblk = pl.BlockSpec((NC, _R, D), lambda i: (0, i, 0))
    return pl.pallas_call(
        _tcc_body,
        grid=(N // _R,),
        in_specs=[pblk] * 4 +
                 [full((1, 64)),
                  full((64, 16)), full((1, 16)), full((1, 16)),
                  full((64, 64)), full((1, 64)),
                  full((64, 16)), full((1, 16)), full((16, C)), full((1, C))],
        out_specs=blk(C),
        out_shape=jax.ShapeDtypeStruct((N, C), jnp.float32),
    )(po1, po2, q2s, q2f,
      p["gcn"]["b2"].reshape(1, 64),
      p["att"]["W1"], p["att"]["b1"].reshape(1, 16), p["att"]["W2"].reshape(1, 16),
      p["res"]["W"], p["res"]["b"].reshape(1, 64),
      p["mlp"]["W1"], p["mlp"]["b1"].reshape(1, 16),
      p["mlp"]["W2"], p["mlp"]["b2"].reshape(1, C))


def kernel(x, sadj, fadj, asadj, afadj, params):
    sadj, fadj = sadj.reshape(2 * E), fadj.reshape(2 * E)
    asadj, afadj = asadj.reshape(2 * E), afadj.reshape(2 * E)
    outs = _tca(x, params)
    h1 = outs[0:4]
    h2 = outs[4:8]
    ss1, sd1, ss2, sd2 = (o.T for o in outs[8:12])
    g1 = outs[12]

    w_pass = _sc_pass(1)
    plain_pass = _sc_pass(0)

    ps = [w_pass(h1[k], ss1[k], sd1[k], asadj) for k in range(4)] + \
         [w_pass(h2[k], ss2[k], sd2[k], afadj) for k in range(4)]
    q1s = plain_pass(g1, sadj)
    q1f = plain_pass(g1, fadj)

    ho1, so1s, so1d, ho2, so2s, so2d, g2s, g2f = _tcb(ps, q1s, q1f, params)

    po1 = w_pass(ho1, so1s.reshape(N), so1d.reshape(N), asadj)
    po2 = w_pass(ho2, so2s.reshape(N), so2d.reshape(N), afadj)
    q2s = plain_pass(g2s, sadj)
    q2f = plain_pass(g2f, fadj)

    return _tcc(po1, po2, q2s, q2f, params)


# speedup vs baseline: 23.9221x; 1.8481x over previous
import functools

import jax
import jax.numpy as jnp
from jax import lax
from jax.experimental import pallas as pl
from jax.experimental.pallas import tpu as pltpu
from jax.experimental.pallas import tpu_sc as plsc

N = 10000
E = 320000
F = 128
C = 16

NC = 2
NS = 16
NW = NC * NS
EPT = E // NW
B = 80
NB = EPT // B
D = 128
SUB_ROWS = 640
SUB_CHUNKS = 8

_R = 1000


def _copy_range(sid, src_at, dst_at):
    for j in range(SUB_CHUNKS):
        start = pl.multiple_of(sid * SUB_ROWS + j * B, 8)

        @pl.when(start < N)
        def _():
            pltpu.sync_copy(src_at(start), dst_at(start))


MEGA = 5
NM = NB // MEGA


@functools.lru_cache(maxsize=None)
def _sc_pass(H):
    mesh = plsc.VectorSubcoreMesh(core_axis_name="c", subcore_axis_name="s",
                                  num_cores=NC, num_subcores=NS)
    scratch = [
        pltpu.VMEM((MEGA * B,), jnp.int32),
        [pltpu.VMEM((B,), jnp.int32) for _ in range(2)],
        [pltpu.VMEM((B, D), jnp.float32) for _ in range(2)],
        pltpu.VMEM_SHARED((N, D), jnp.float32),
        [pltpu.SemaphoreType.DMA for _ in range(4)],
    ]
    if H:
        scratch += [
            pltpu.VMEM((N,), jnp.float32),
            pltpu.VMEM((N,), jnp.float32),
            pltpu.VMEM((B,), jnp.float32),
        ]

    def body(rows_hbm, ssrc_hbm, sdst_hbm, adj_hbm, out_hbm,
             src_m, dst_vs, rows_vs, acc, sems, *wscratch):
        cid = lax.axis_index("c")
        sid = lax.axis_index("s")
        wid = cid * NS + sid
        rows_v = rows_vs[0]

        def zrow(i, carry):
            for j in range(D // 16):
                rows_v[i, pl.ds(j * 16, 16)] = jnp.zeros((16,), jnp.float32)
            return carry
        lax.fori_loop(0, B, zrow, 0)
        _copy_range(sid, lambda s: rows_v, lambda s: acc.at[pl.ds(s, B)])

        if H:
            s_src_v, s_dst_v, wbuf = wscratch
            pltpu.sync_copy(ssrc_hbm, s_src_v)
            pltpu.sync_copy(sdst_hbm, s_dst_v)
        plsc.subcore_barrier()

        def issue(mbase, j):
            jb = j % 2
            pltpu.async_copy(adj_hbm.at[pl.ds(E + mbase + j * B, B)],
                             dst_vs[jb], sems[jb])
            pltpu.async_copy(rows_hbm.at[src_m.at[pl.ds(j * B, B)]],
                             rows_vs[jb], sems[2 + jb])

        def mega(m, carry):
            mbase = wid * EPT + m * (MEGA * B)
            pltpu.sync_copy(adj_hbm.at[pl.ds(mbase, MEGA * B)], src_m)
            issue(mbase, 0)
            for j in range(MEGA):
                jb = j % 2
                dst_v, rows_v = dst_vs[jb], rows_vs[jb]
                pltpu.make_async_copy(adj_hbm.at[pl.ds(E + mbase + j * B, B)],
                                      dst_v, sems[jb]).wait()
                pltpu.make_async_copy(rows_hbm.at[src_m.at[pl.ds(j * B, B)]],
                                      rows_v, sems[2 + jb]).wait()
                if j + 1 < MEGA:
                    issue(mbase, j + 1)
                if H:
                    for e0 in range(0, B, 16):
                        idxs = src_m[pl.ds(j * B + e0, 16)]
                        idxd = dst_v[pl.ds(e0, 16)]
                        ev = (plsc.load_gather(s_src_v, [idxs])
                              + plsc.load_gather(s_dst_v, [idxd]))
                        ev = jnp.where(ev >= 0.0, ev, 0.2 * ev)
                        wbuf[pl.ds(e0, 16)] = jnp.exp(ev)

                    def scale(i, carry):
                        for u in range(4):
                            e = i * 4 + u
                            w = plsc.load_gather(wbuf, [jnp.full((16,), e, jnp.int32)])
                            for jj in range(5):
                                sl = pl.ds(jj * 16, 16)
                                rows_v[e, sl] = rows_v[e, sl] * w
                        return carry
                    lax.fori_loop(0, B // 4, scale, 0)
                pltpu.sync_copy(rows_v, acc.at[dst_v], add=True)
            return carry
        lax.fori_loop(0, NM, mega, 0)
        plsc.subcore_barrier()

        _copy_range(sid, lambda s: acc.at[pl.ds(s, B)],
                    lambda s: out_hbm.at[cid, pl.ds(s, B)])

    cp = pltpu.CompilerParams(needs_layout_passes=False)
    if H:
        return pl.kernel(body,
                         out_type=jax.ShapeDtypeStruct((NC, N, D), jnp.float32),
                         mesh=mesh, scratch_types=scratch, compiler_params=cp)
    def body0(rows_hbm, adj_hbm, out_hbm, *rest):
        return body(rows_hbm, None, None, adj_hbm, out_hbm, *rest)
    return pl.kernel(body0,
                     out_type=jax.ShapeDtypeStruct((NC, N, D), jnp.float32),
                     mesh=mesh, scratch_types=scratch, compiler_params=cp)


def _ones_pad(ref, r):
    lane = lax.broadcasted_iota(jnp.int32, (r, 16), 1)
    ref[:, 64:80] = (lane < 1).astype(jnp.float32)
    ref[:, 80:128] = jnp.zeros((r, 48), jnp.float32)


def _tca_body(x_ref, *refs):
    (w10, w11, w12, w13, w20, w21, w22, w23, as1_ref, ad1_ref, as2_ref, ad2_ref,
     wg_ref,
     h10, h11, h12, h13, h20, h21, h22, h23,
     ss1_ref, sd1_ref, ss2_ref, sd2_ref, g1_ref) = refs
    x = x_ref[...]
    for (ws, hrefs, ss_ref, sd_ref, as_ref, ad_ref) in [
            ((w10, w11, w12, w13), (h10, h11, h12, h13), ss1_ref, sd1_ref, as1_ref, ad1_ref),
            ((w20, w21, w22, w23), (h20, h21, h22, h23), ss2_ref, sd2_ref, as2_ref, ad2_ref)]:
        for k in range(4):
            h = jnp.dot(x, ws[k][...], preferred_element_type=jnp.float32)
            hrefs[k][:, 0:64] = h
            _ones_pad(hrefs[k], _R)
            ss_ref[:, k:k + 1] = jnp.sum(h * as_ref[k:k + 1, :], axis=1, keepdims=True)
            sd_ref[:, k:k + 1] = jnp.sum(h * ad_ref[k:k + 1, :], axis=1, keepdims=True)
    g1_ref[:, 0:64] = jnp.dot(x, wg_ref[...], preferred_element_type=jnp.float32)
    g1_ref[:, 64:128] = jnp.zeros((_R, 64), jnp.float32)


def _tca(x, p):
    full = lambda shape: pl.BlockSpec(shape, lambda i: (0,) * len(shape))
    blk = lambda w: pl.BlockSpec((_R, w), lambda i: (i, 0))
    sblk = pl.BlockSpec((_R, 4), lambda i: (i, 0))
    g1h, g2h = p["gat1"]["heads"], p["gat2"]["heads"]
    asrc1 = jnp.stack([h["a_src"] for h in g1h])
    adst1 = jnp.stack([h["a_dst"] for h in g1h])
    asrc2 = jnp.stack([h["a_src"] for h in g2h])
    adst2 = jnp.stack([h["a_dst"] for h in g2h])
    outs = [jax.ShapeDtypeStruct((N, D), jnp.float32)] * 8 + \
           [jax.ShapeDtypeStruct((N, 4), jnp.float32)] * 4 + \
           [jax.ShapeDtypeStruct((N, D), jnp.float32)]
    return pl.pallas_call(
        _tca_body,
        grid=(N // _R,),
        in_specs=[blk(F)] + [full((F, 64))] * 8 + [full((4, 64))] * 4 + [full((F, 64))],
        out_specs=[blk(D)] * 8 + [sblk] * 4 + [blk(D)],
        out_shape=outs,
    )(x, *[h["W"] for h in g1h], *[h["W"] for h in g2h],
      asrc1, adst1, asrc2, adst2, p["gcn"]["W1"])


def _norm(pref):
    acc = pref[0] + pref[1]
    return acc[:, 0:64] / (acc[:, 64:65] + 1e-16)


def _tcb_body(p10, p11, p12, p13, p20, p21, p22, p23, q1s_ref, q1f_ref,
              wo1_ref, as1_ref, ad1_ref, wo2_ref, as2_ref, ad2_ref,
              b1_ref, w2_ref,
              ho1_ref, so1s_ref, so1d_ref, ho2_ref, so2s_ref, so2d_ref,
              g2s_ref, g2f_ref):
    for (prefs, wo_ref, as_ref, ad_ref, ho_ref, sos_ref, sod_ref) in [
            ((p10, p11, p12, p13), wo1_ref, as1_ref, ad1_ref, ho1_ref, so1s_ref, so1d_ref),
            ((p20, p21, p22, p23), wo2_ref, as2_ref, ad2_ref, ho2_ref, so2s_ref, so2d_ref)]:
        hcat = jnp.concatenate([_norm(p) for p in prefs], axis=1)
        ho = jnp.dot(hcat, wo_ref[...], preferred_element_type=jnp.float32)
        ho_ref[:, 0:64] = ho
        _ones_pad(ho_ref, _R)
        sos_ref[...] = jnp.sum(ho * as_ref[...], axis=1, keepdims=True)
        sod_ref[...] = jnp.sum(ho * ad_ref[...], axis=1, keepdims=True)
    b1 = b1_ref[...]
    w2 = w2_ref[...]
    zero = jnp.zeros((_R, 64), jnp.float32)
    hs = jax.nn.relu(q1s_ref[0, :, 0:64] + q1s_ref[1, :, 0:64] + b1)
    hf = jax.nn.relu(q1f_ref[0, :, 0:64] + q1f_ref[1, :, 0:64] + b1)
    g2s_ref[:, 0:64] = jnp.dot(hs, w2, preferred_element_type=jnp.float32)
    g2s_ref[:, 64:128] = zero
    g2f_ref[:, 0:64] = jnp.dot(hf, w2, preferred_element_type=jnp.float32)
    g2f_ref[:, 64:128] = zero


def _tcb(ps, q1s, q1f, p):
    full = lambda shape: pl.BlockSpec(shape, lambda i: (0,) * len(shape))
    blk = lambda w: pl.BlockSpec((_R, w), lambda i: (i, 0))
    pblk = pl.BlockSpec((NC, _R, D), lambda i: (0, i, 0))
    sblk = pl.BlockSpec((_R, 1), lambda i: (i, 0))
    outs = [jax.ShapeDtypeStruct((N, D), jnp.float32),
            jax.ShapeDtypeStruct((N, 1), jnp.float32),
            jax.ShapeDtypeStruct((N, 1), jnp.float32)] * 2 + \
           [jax.ShapeDtypeStruct((N, D), jnp.float32)] * 2
    o1, o2 = p["gat1"]["out"], p["gat2"]["out"]
    return pl.pallas_call(
        _tcb_body,
        grid=(N // _R,),
        in_specs=[pblk] * 10 +
                 [full((256, 64)), full((1, 64)), full((1, 64))] * 2 +
                 [full((1, 64)), full((64, 64))],
        out_specs=[blk(D), sblk, sblk, blk(D), sblk, sblk, blk(D), blk(D)],
        out_shape=outs,
    )(*ps, q1s, q1f,
      o1["W"], o1["a_src"].reshape(1, 64), o1["a_dst"].reshape(1, 64),
      o2["W"], o2["a_src"].reshape(1, 64), o2["a_dst"].reshape(1, 64),
      p["gcn"]["b1"].reshape(1, 64), p["gcn"]["W2"])


def _tcc_body(po1_ref, po2_ref, q2s_ref, q2f_ref,
              b2_ref, aW1_ref, ab1_ref, aW2_ref, rW_ref, rb_ref,
              mW1_ref, mb1_ref, mW2_ref, mb2_ref, out_ref):
    def emb_of(po_ref):
        h = _norm(po_ref)
        h = jnp.where(h > 0.0, h, jnp.exp(jnp.minimum(h, 0.0)) - 1.0)
        m = jnp.max(h, axis=1, keepdims=True)
        ex = jnp.exp(h - m)
        return ex / jnp.sum(ex, axis=1, keepdims=True)

    e1 = emb_of(po1_ref)
    e2 = emb_of(po2_ref)
    b2 = b2_ref[...]
    com1 = jax.nn.relu(q2s_ref[0, :, 0:64] + q2s_ref[1, :, 0:64] + b2)
    com2 = jax.nn.relu(q2f_ref[0, :, 0:64] + q2f_ref[1, :, 0:64] + b2)
    xc = (com1 + com2) * 0.5

    aW1, ab1, aW2 = aW1_ref[...], ab1_ref[...], aW2_ref[...]

    def att_w(e):
        t = jnp.tanh(jnp.dot(e, aW1, preferred_element_type=jnp.float32) + ab1)
        return jnp.sum(t * aW2, axis=1, keepdims=True)

    w1, w2, w3 = att_w(e1), att_w(e2), att_w(xc)
    m = jnp.maximum(jnp.maximum(w1, w2), w3)
    x1, x2, x3 = jnp.exp(w1 - m), jnp.exp(w2 - m), jnp.exp(w3 - m)
    emb = (x1 * e1 + x2 * e2 + x3 * xc) / (x1 + x2 + x3)
    emb = emb + jnp.dot(emb, rW_ref[...], preferred_element_type=jnp.float32) + rb_ref[...]
    h = jnp.tanh(jnp.dot(emb, mW1_ref[...], preferred_element_type=jnp.float32) + mb1_ref[...])
    logits = jnp.dot(h, mW2_ref[...], preferred_element_type=jnp.float32) + mb2_ref[...]
    mx = jnp.max(logits, axis=1, keepdims=True)
    lse = jnp.log(jnp.sum(jnp.exp(logits - mx), axis=1, keepdims=True)) + mx
    out_ref[...] = logits - lse


def _tcc(po1, po2, q2s, q2f, p):
    full = lambda shape: pl.BlockSpec(shape, lambda i: (0,) * len(shape))
    blk = lambda w: pl.BlockSpec((_R, w), lambda i: (i, 0))
    pblk = pl.BlockSpec((NC, _R, D), lambda i: (0, i, 0))
    return pl.pallas_call(
        _tcc_body,
        grid=(N // _R,),
        in_specs=[pblk] * 4 +
                 [full((1, 64)),
                  full((64, 16)), full((1, 16)), full((1, 16)),
                  full((64, 64)), full((1, 64)),
                  full((64, 16)), full((1, 16)), full((16, C)), full((1, C))],
        out_specs=blk(C),
        out_shape=jax.ShapeDtypeStruct((N, C), jnp.float32),
    )(po1, po2, q2s, q2f,
      p["gcn"]["b2"].reshape(1, 64),
      p["att"]["W1"], p["att"]["b1"].reshape(1, 16), p["att"]["W2"].reshape(1, 16),
      p["res"]["W"], p["res"]["b"].reshape(1, 64),
      p["mlp"]["W1"], p["mlp"]["b1"].reshape(1, 16),
      p["mlp"]["W2"], p["mlp"]["b2"].reshape(1, C))


def kernel(x, sadj, fadj, asadj, afadj, params):
    sadj, fadj = sadj.reshape(2 * E), fadj.reshape(2 * E)
    asadj, afadj = asadj.reshape(2 * E), afadj.reshape(2 * E)
    outs = _tca(x, params)
    h1 = outs[0:4]
    h2 = outs[4:8]
    ss1, sd1, ss2, sd2 = (o.T for o in outs[8:12])
    g1 = outs[12]

    w_pass = _sc_pass(1)
    plain_pass = _sc_pass(0)

    ps = [w_pass(h1[k], ss1[k], sd1[k], asadj) for k in range(4)] + \
         [w_pass(h2[k], ss2[k], sd2[k], afadj) for k in range(4)]
    q1s = plain_pass(g1, sadj)
    q1f = plain_pass(g1, fadj)

    ho1, so1s, so1d, ho2, so2s, so2d, g2s, g2f = _tcb(ps, q1s, q1f, params)

    po1 = w_pass(ho1, so1s.reshape(N), so1d.reshape(N), asadj)
    po2 = w_pass(ho2, so2s.reshape(N), so2d.reshape(N), afadj)
    q2s = plain_pass(g2s, sadj)
    q2f = plain_pass(g2f, fadj)

    return _tcc(po1, po2, q2s, q2f, params)


# speedup vs baseline: 29.0989x; 1.2164x over previous
import functools

import jax
import jax.numpy as jnp
from jax import lax
from jax.experimental import pallas as pl
from jax.experimental.pallas import tpu as pltpu
from jax.experimental.pallas import tpu_sc as plsc

N = 10000
E = 320000
F = 128
C = 16

NC = 2
NS = 16
NW = NC * NS
EPT = E // NW
B = 80
NB = EPT // B
D = 128
SUB_ROWS = 640
SUB_CHUNKS = 8

_R = 1000


def _copy_range(sid, src_at, dst_at):
    for j in range(SUB_CHUNKS):
        start = pl.multiple_of(sid * SUB_ROWS + j * B, 8)

        @pl.when(start < N)
        def _():
            pltpu.sync_copy(src_at(start), dst_at(start))


MEGA = 5
NM = NB // MEGA
DR = 160


@functools.lru_cache(maxsize=None)
def _sc_wpass():
    mesh = plsc.VectorSubcoreMesh(core_axis_name="c", subcore_axis_name="s",
                                  num_cores=NC, num_subcores=NS)
    scratch = [
        pltpu.VMEM((MEGA * B,), jnp.int32),
        pltpu.VMEM((MEGA * B,), jnp.int32),
        [pltpu.VMEM((B,), jnp.float32) for _ in range(2)],
        [pltpu.VMEM((B,), jnp.float32) for _ in range(2)],
        pltpu.VMEM((DR, 128), jnp.float32),
        pltpu.VMEM((2, B), jnp.int32),
        pltpu.VMEM_SHARED((DR, 128), jnp.float32),
        [pltpu.SemaphoreType.DMA for _ in range(2)],
        pltpu.VMEM((N,), jnp.float32),
        pltpu.VMEM((N,), jnp.float32),
        pltpu.VMEM((N,), jnp.float32),
        pltpu.VMEM((N,), jnp.float32),
    ]

    def body(ssrc0_hbm, sdst0_hbm, ssrc1_hbm, sdst1_hbm, adj_hbm,
             w_hbm, den_hbm,
             src_m, dst_m, wb0s, wb1s, den_v, rid_v, denacc, sems,
             s0s, s0d, s1s, s1d):
        cid = lax.axis_index("c")
        sid = lax.axis_index("s")
        wid = cid * NS + sid

        def zden(i, carry):
            for j in range(8):
                den_v[i, pl.ds(j * 16, 16)] = jnp.zeros((16,), jnp.float32)
            return carry
        lax.fori_loop(0, DR, zden, 0)
        for r in range(2):
            for j in range(B // 16):
                rid_v[r, pl.ds(j * 16, 16)] = (
                    lax.iota(jnp.int32, 16) + (r * B + j * 16))
        pltpu.sync_copy(ssrc0_hbm, s0s)
        pltpu.sync_copy(sdst0_hbm, s0d)
        pltpu.sync_copy(ssrc1_hbm, s1s)
        pltpu.sync_copy(sdst1_hbm, s1d)

        @pl.when(sid == 0)
        def _():
            pltpu.sync_copy(den_v, denacc)
        plsc.subcore_barrier()

        def wcalc(idxs, idxd, ss, sd):
            ev = plsc.load_gather(ss, [idxs]) + plsc.load_gather(sd, [idxd])
            ev = jnp.where(ev >= 0.0, ev, 0.2 * ev)
            return jnp.exp(ev)

        def mega(m, carry):
            mbase = wid * EPT + m * (MEGA * B)
            pltpu.sync_copy(adj_hbm.at[pl.ds(mbase, MEGA * B)], src_m)
            pltpu.sync_copy(adj_hbm.at[pl.ds(E + mbase, MEGA * B)], dst_m)
            for j in range(MEGA):
                jb = j % 2
                wb0, wb1 = wb0s[jb], wb1s[jb]
                base = mbase + j * B

                @pl.when(m * MEGA + j >= 2)
                def _():
                    pltpu.make_async_copy(
                        wb0, w_hbm.at[pl.ds(base, B)], sems[jb]).wait()
                    pltpu.make_async_copy(
                        wb1, w_hbm.at[pl.ds(E + base, B)], sems[jb]).wait()
                for e0 in range(0, B, 16):
                    idxs = src_m[pl.ds(j * B + e0, 16)]
                    idxd = dst_m[pl.ds(j * B + e0, 16)]
                    flat = idxd * 2
                    w0 = wcalc(idxs, idxd, s0s, s0d)
                    wb0[pl.ds(e0, 16)] = w0
                    plsc.addupdate_scatter(
                        den_v, [lax.shift_right_logical(flat, 7),
                                lax.bitwise_and(flat, 127)], w0)
                    w1 = wcalc(idxs, idxd, s1s, s1d)
                    wb1[pl.ds(e0, 16)] = w1
                    plsc.addupdate_scatter(
                        den_v, [lax.shift_right_logical(flat + 1, 7),
                                lax.bitwise_and(flat + 1, 127)], w1)
                pltpu.async_copy(wb0, w_hbm.at[pl.ds(base, B)], sems[jb])
                pltpu.async_copy(wb1, w_hbm.at[pl.ds(E + base, B)], sems[jb])
            return carry
        lax.fori_loop(0, NM, mega, 0)
        for jb in range(2):
            pltpu.make_async_copy(wb0s[jb], w_hbm.at[pl.ds(0, B)], sems[jb]).wait()
            pltpu.make_async_copy(wb1s[jb], w_hbm.at[pl.ds(0, B)], sems[jb]).wait()

        pltpu.sync_copy(den_v.at[pl.ds(0, B)], denacc.at[rid_v.at[0]], add=True)
        pltpu.sync_copy(den_v.at[pl.ds(B, B)], denacc.at[rid_v.at[1]], add=True)
        plsc.subcore_barrier()

        @pl.when(sid == 0)
        def _():
            pltpu.sync_copy(denacc, den_hbm.at[cid])

    cp = pltpu.CompilerParams(needs_layout_passes=False)
    return pl.kernel(
        body,
        out_type=(jax.ShapeDtypeStruct((2 * E,), jnp.float32),
                  jax.ShapeDtypeStruct((NC, DR, 128), jnp.float32)),
        mesh=mesh, scratch_types=scratch, compiler_params=cp)


@functools.lru_cache(maxsize=None)
def _sc_pair():
    mesh = plsc.VectorSubcoreMesh(core_axis_name="c", subcore_axis_name="s",
                                  num_cores=NC, num_subcores=NS)
    scratch = [
        pltpu.VMEM((MEGA * B,), jnp.int32),
        [pltpu.VMEM((B,), jnp.int32) for _ in range(2)],
        [pltpu.VMEM((B, D), jnp.float32) for _ in range(2)],
        [pltpu.VMEM((B,), jnp.float32) for _ in range(2)],
        [pltpu.VMEM((B,), jnp.float32) for _ in range(2)],
        pltpu.VMEM_SHARED((N, D), jnp.float32),
        [pltpu.SemaphoreType.DMA for _ in range(6)],
    ]

    def body(rows_hbm, w_hbm, adj_hbm, out_hbm,
             src_m, dst_vs, rows_vs, wb0s, wb1s, acc, sems):
        cid = lax.axis_index("c")
        sid = lax.axis_index("s")
        wid = cid * NS + sid
        rows_v = rows_vs[0]

        def zrow(i, carry):
            for j in range(D // 16):
                rows_v[i, pl.ds(j * 16, 16)] = jnp.zeros((16,), jnp.float32)
            return carry
        lax.fori_loop(0, B, zrow, 0)
        _copy_range(sid, lambda s: rows_v, lambda s: acc.at[pl.ds(s, B)])
        plsc.subcore_barrier()

        def issue(mbase, j):
            jb = j % 2
            pltpu.async_copy(adj_hbm.at[pl.ds(E + mbase + j * B, B)],
                             dst_vs[jb], sems[jb])
            pltpu.async_copy(rows_hbm.at[src_m.at[pl.ds(j * B, B)]],
                             rows_vs[jb], sems[2 + jb])
            pltpu.async_copy(w_hbm.at[pl.ds(mbase + j * B, B)],
                             wb0s[jb], sems[4 + jb])
            pltpu.async_copy(w_hbm.at[pl.ds(E + mbase + j * B, B)],
                             wb1s[jb], sems[4 + jb])

        def mega(m, carry):
            mbase = wid * EPT + m * (MEGA * B)
            pltpu.sync_copy(adj_hbm.at[pl.ds(mbase, MEGA * B)], src_m)
            issue(mbase, 0)
            for j in range(MEGA):
                jb = j % 2
                dst_v, rows_v = dst_vs[jb], rows_vs[jb]
                wb0, wb1 = wb0s[jb], wb1s[jb]
                pltpu.make_async_copy(adj_hbm.at[pl.ds(E + mbase + j * B, B)],
                                      dst_v, sems[jb]).wait()
                pltpu.make_async_copy(rows_hbm.at[src_m.at[pl.ds(j * B, B)]],
                                      rows_v, sems[2 + jb]).wait()
                pltpu.make_async_copy(w_hbm.at[pl.ds(mbase + j * B, B)],
                                      wb0, sems[4 + jb]).wait()
                pltpu.make_async_copy(w_hbm.at[pl.ds(E + mbase + j * B, B)],
                                      wb1, sems[4 + jb]).wait()
                if j + 1 < MEGA:
                    issue(mbase, j + 1)

                def scale(i, carry):
                    for u in range(2):
                        e = i * 2 + u
                        ee = jnp.full((16,), e, jnp.int32)
                        w0 = plsc.load_gather(wb0, [ee])
                        w1 = plsc.load_gather(wb1, [ee])
                        for jj in range(4):
                            sl = pl.ds(jj * 16, 16)
                            rows_v[e, sl] = rows_v[e, sl] * w0
                        for jj in range(4, 8):
                            sl = pl.ds(jj * 16, 16)
                            rows_v[e, sl] = rows_v[e, sl] * w1
                    return carry
                lax.fori_loop(0, B // 2, scale, 0)
                pltpu.sync_copy(rows_v, acc.at[dst_v], add=True)
            return carry
        lax.fori_loop(0, NM, mega, 0)
        plsc.subcore_barrier()

        _copy_range(sid, lambda s: acc.at[pl.ds(s, B)],
                    lambda s: out_hbm.at[cid, pl.ds(s, B)])

    cp = pltpu.CompilerParams(needs_layout_passes=False)
    return pl.kernel(
        body,
        out_type=jax.ShapeDtypeStruct((NC, N, D), jnp.float32),
        mesh=mesh, scratch_types=scratch, compiler_params=cp)


@functools.lru_cache(maxsize=None)
def _sc_pass(H):
    mesh = plsc.VectorSubcoreMesh(core_axis_name="c", subcore_axis_name="s",
                                  num_cores=NC, num_subcores=NS)
    scratch = [
        pltpu.VMEM((MEGA * B,), jnp.int32),
        [pltpu.VMEM((B,), jnp.int32) for _ in range(2)],
        [pltpu.VMEM((B, D), jnp.float32) for _ in range(2)],
        pltpu.VMEM_SHARED((N, D), jnp.float32),
        [pltpu.SemaphoreType.DMA for _ in range(4)],
    ]
    if H:
        scratch += [
            pltpu.VMEM((N,), jnp.float32),
            pltpu.VMEM((N,), jnp.float32),
            pltpu.VMEM((B,), jnp.float32),
        ]

    def body(rows_hbm, ssrc_hbm, sdst_hbm, adj_hbm, out_hbm,
             src_m, dst_vs, rows_vs, acc, sems, *wscratch):
        cid = lax.axis_index("c")
        sid = lax.axis_index("s")
        wid = cid * NS + sid
        rows_v = rows_vs[0]

        def zrow(i, carry):
            for j in range(D // 16):
                rows_v[i, pl.ds(j * 16, 16)] = jnp.zeros((16,), jnp.float32)
            return carry
        lax.fori_loop(0, B, zrow, 0)
        _copy_range(sid, lambda s: rows_v, lambda s: acc.at[pl.ds(s, B)])

        if H:
            s_src_v, s_dst_v, wbuf = wscratch
            pltpu.sync_copy(ssrc_hbm, s_src_v)
            pltpu.sync_copy(sdst_hbm, s_dst_v)
        plsc.subcore_barrier()

        def issue(mbase, j):
            jb = j % 2
            pltpu.async_copy(adj_hbm.at[pl.ds(E + mbase + j * B, B)],
                             dst_vs[jb], sems[jb])
            pltpu.async_copy(rows_hbm.at[src_m.at[pl.ds(j * B, B)]],
                             rows_vs[jb], sems[2 + jb])

        def mega(m, carry):
            mbase = wid * EPT + m * (MEGA * B)
            pltpu.sync_copy(adj_hbm.at[pl.ds(mbase, MEGA * B)], src_m)
            issue(mbase, 0)
            for j in range(MEGA):
                jb = j % 2
                dst_v, rows_v = dst_vs[jb], rows_vs[jb]
                pltpu.make_async_copy(adj_hbm.at[pl.ds(E + mbase + j * B, B)],
                                      dst_v, sems[jb]).wait()
                pltpu.make_async_copy(rows_hbm.at[src_m.at[pl.ds(j * B, B)]],
                                      rows_v, sems[2 + jb]).wait()
                if j + 1 < MEGA:
                    issue(mbase, j + 1)
                if H:
                    for e0 in range(0, B, 16):
                        idxs = src_m[pl.ds(j * B + e0, 16)]
                        idxd = dst_v[pl.ds(e0, 16)]
                        ev = (plsc.load_gather(s_src_v, [idxs])
                              + plsc.load_gather(s_dst_v, [idxd]))
                        ev = jnp.where(ev >= 0.0, ev, 0.2 * ev)
                        wbuf[pl.ds(e0, 16)] = jnp.exp(ev)

                    def scale(i, carry):
                        for u in range(4):
                            e = i * 4 + u
                            w = plsc.load_gather(wbuf, [jnp.full((16,), e, jnp.int32)])
                            for jj in range(5):
                                sl = pl.ds(jj * 16, 16)
                                rows_v[e, sl] = rows_v[e, sl] * w
                        return carry
                    lax.fori_loop(0, B // 4, scale, 0)
                pltpu.sync_copy(rows_v, acc.at[dst_v], add=True)
            return carry
        lax.fori_loop(0, NM, mega, 0)
        plsc.subcore_barrier()

        _copy_range(sid, lambda s: acc.at[pl.ds(s, B)],
                    lambda s: out_hbm.at[cid, pl.ds(s, B)])

    cp = pltpu.CompilerParams(needs_layout_passes=False)
    if H:
        return pl.kernel(body,
                         out_type=jax.ShapeDtypeStruct((NC, N, D), jnp.float32),
                         mesh=mesh, scratch_types=scratch, compiler_params=cp)
    def body0(rows_hbm, adj_hbm, out_hbm, *rest):
        return body(rows_hbm, None, None, adj_hbm, out_hbm, *rest)
    return pl.kernel(body0,
                     out_type=jax.ShapeDtypeStruct((NC, N, D), jnp.float32),
                     mesh=mesh, scratch_types=scratch, compiler_params=cp)


def _ones_pad(ref, r):
    lane = lax.broadcasted_iota(jnp.int32, (r, 16), 1)
    ref[:, 64:80] = (lane < 1).astype(jnp.float32)
    ref[:, 80:128] = jnp.zeros((r, 48), jnp.float32)


def _tca_body(x_ref, *refs):
    (w10, w11, w12, w13, w20, w21, w22, w23, as1_ref, ad1_ref, as2_ref, ad2_ref,
     wg_ref,
     pa1, pb1, pa2, pb2,
     ss1_ref, sd1_ref, ss2_ref, sd2_ref, g1_ref) = refs
    x = x_ref[...]
    for (ws, prefs, ss_ref, sd_ref, as_ref, ad_ref) in [
            ((w10, w11, w12, w13), (pa1, pb1), ss1_ref, sd1_ref, as1_ref, ad1_ref),
            ((w20, w21, w22, w23), (pa2, pb2), ss2_ref, sd2_ref, as2_ref, ad2_ref)]:
        for k in range(4):
            h = jnp.dot(x, ws[k][...], preferred_element_type=jnp.float32)
            c = (k % 2) * 64
            prefs[k // 2][:, c:c + 64] = h
            ss_ref[:, k:k + 1] = jnp.sum(h * as_ref[k:k + 1, :], axis=1, keepdims=True)
            sd_ref[:, k:k + 1] = jnp.sum(h * ad_ref[k:k + 1, :], axis=1, keepdims=True)
    g1_ref[:, 0:64] = jnp.dot(x, wg_ref[...], preferred_element_type=jnp.float32)
    g1_ref[:, 64:128] = jnp.zeros((_R, 64), jnp.float32)


def _tca(x, p):
    full = lambda shape: pl.BlockSpec(shape, lambda i: (0,) * len(shape))
    blk = lambda w: pl.BlockSpec((_R, w), lambda i: (i, 0))
    sblk = pl.BlockSpec((_R, 4), lambda i: (i, 0))
    g1h, g2h = p["gat1"]["heads"], p["gat2"]["heads"]
    asrc1 = jnp.stack([h["a_src"] for h in g1h])
    adst1 = jnp.stack([h["a_dst"] for h in g1h])
    asrc2 = jnp.stack([h["a_src"] for h in g2h])
    adst2 = jnp.stack([h["a_dst"] for h in g2h])
    outs = [jax.ShapeDtypeStruct((N, D), jnp.float32)] * 4 + \
           [jax.ShapeDtypeStruct((N, 4), jnp.float32)] * 4 + \
           [jax.ShapeDtypeStruct((N, D), jnp.float32)]
    return pl.pallas_call(
        _tca_body,
        grid=(N // _R,),
        in_specs=[blk(F)] + [full((F, 64))] * 8 + [full((4, 64))] * 4 + [full((F, 64))],
        out_specs=[blk(D)] * 4 + [sblk] * 4 + [blk(D)],
        out_shape=outs,
    )(x, *[h["W"] for h in g1h], *[h["W"] for h in g2h],
      asrc1, adst1, asrc2, adst2, p["gcn"]["W1"])


def _norm(pref):
    acc = pref[0] + pref[1]
    return acc[:, 0:64] / (acc[:, 64:65] + 1e-16)


def _tcb_body(p1a, p1b, p2a, p2b, d1_ref, d2_ref, q1s_ref, q1f_ref,
              wo1_ref, as1_ref, ad1_ref, wo2_ref, as2_ref, ad2_ref,
              b1_ref, w2_ref,
              ho1_ref, so1s_ref, so1d_ref, ho2_ref, so2s_ref, so2d_ref,
              g2s_ref, g2f_ref):
    for (pa, pb, d_ref, wo_ref, as_ref, ad_ref, ho_ref, sos_ref, sod_ref) in [
            (p1a, p1b, d1_ref, wo1_ref, as1_ref, ad1_ref, ho1_ref, so1s_ref, so1d_ref),
            (p2a, p2b, d2_ref, wo2_ref, as2_ref, ad2_ref, ho2_ref, so2s_ref, so2d_ref)]:
        acca = pa[0] + pa[1]
        accb = pb[0] + pb[1]
        hcat = jnp.concatenate(
            [acca[:, 0:64] / (d_ref[:, 0:1] + 1e-16),
             acca[:, 64:128] / (d_ref[:, 1:2] + 1e-16),
             accb[:, 0:64] / (d_ref[:, 2:3] + 1e-16),
             accb[:, 64:128] / (d_ref[:, 3:4] + 1e-16)], axis=1)
        ho = jnp.dot(hcat, wo_ref[...], preferred_element_type=jnp.float32)
        ho_ref[:, 0:64] = ho
        _ones_pad(ho_ref, _R)
        sos_ref[...] = jnp.sum(ho * as_ref[...], axis=1, keepdims=True)
        sod_ref[...] = jnp.sum(ho * ad_ref[...], axis=1, keepdims=True)
    b1 = b1_ref[...]
    w2 = w2_ref[...]
    zero = jnp.zeros((_R, 64), jnp.float32)
    hs = jax.nn.relu(q1s_ref[0, :, 0:64] + q1s_ref[1, :, 0:64] + b1)
    hf = jax.nn.relu(q1f_ref[0, :, 0:64] + q1f_ref[1, :, 0:64] + b1)
    g2s_ref[:, 0:64] = jnp.dot(hs, w2, preferred_element_type=jnp.float32)
    g2s_ref[:, 64:128] = zero
    g2f_ref[:, 0:64] = jnp.dot(hf, w2, preferred_element_type=jnp.float32)
    g2f_ref[:, 64:128] = zero


def _tcb(ps, d1, d2, q1s, q1f, p):
    full = lambda shape: pl.BlockSpec(shape, lambda i: (0,) * len(shape))
    blk = lambda w: pl.BlockSpec((_R, w), lambda i: (i, 0))
    pblk = pl.BlockSpec((NC, _R, D), lambda i: (0, i, 0))
    sblk = pl.BlockSpec((_R, 1), lambda i: (i, 0))
    dblk = pl.BlockSpec((_R, 4), lambda i: (i, 0))
    outs = [jax.ShapeDtypeStruct((N, D), jnp.float32),
            jax.ShapeDtypeStruct((N, 1), jnp.float32),
            jax.ShapeDtypeStruct((N, 1), jnp.float32)] * 2 + \
           [jax.ShapeDtypeStruct((N, D), jnp.float32)] * 2
    o1, o2 = p["gat1"]["out"], p["gat2"]["out"]
    return pl.pallas_call(
        _tcb_body,
        grid=(N // _R,),
        in_specs=[pblk] * 4 + [dblk] * 2 + [pblk] * 2 +
                 [full((256, 64)), full((1, 64)), full((1, 64))] * 2 +
                 [full((1, 64)), full((64, 64))],
        out_specs=[blk(D), sblk, sblk, blk(D), sblk, sblk, blk(D), blk(D)],
        out_shape=outs,
    )(*ps, d1, d2, q1s, q1f,
      o1["W"], o1["a_src"].reshape(1, 64), o1["a_dst"].reshape(1, 64),
      o2["W"], o2["a_src"].reshape(1, 64), o2["a_dst"].reshape(1, 64),
      p["gcn"]["b1"].reshape(1, 64), p["gcn"]["W2"])


def _tcc_body(po1_ref, po2_ref, q2s_ref, q2f_ref,
              b2_ref, aW1_ref, ab1_ref, aW2_ref, rW_ref, rb_ref,
              mW1_ref, mb1_ref, mW2_ref, mb2_ref, out_ref):
    def emb_of(po_ref):
        h = _norm(po_ref)
        h = jnp.where(h > 0.0, h, jnp.exp(jnp.minimum(h, 0.0)) - 1.0)
        m = jnp.max(h, axis=1, keepdims=True)
        ex = jnp.exp(h - m)
        return ex / jnp.sum(ex, axis=1, keepdims=True)

    e1 = emb_of(po1_ref)
    e2 = emb_of(po2_ref)
    b2 = b2_ref[...]
    com1 = jax.nn.relu(q2s_ref[0, :, 0:64] + q2s_ref[1, :, 0:64] + b2)
    com2 = jax.nn.relu(q2f_ref[0, :, 0:64] + q2f_ref[1, :, 0:64] + b2)
    xc = (com1 + com2) * 0.5

    aW1, ab1, aW2 = aW1_ref[...], ab1_ref[...], aW2_ref[...]

    def att_w(e):
        t = jnp.tanh(jnp.dot(e, aW1, preferred_element_type=jnp.float32) + ab1)
        return jnp.sum(t * aW2, axis=1, keepdims=True)

    w1, w2, w3 = att_w(e1), att_w(e2), att_w(xc)
    m = jnp.maximum(jnp.maximum(w1, w2), w3)
    x1, x2, x3 = jnp.exp(w1 - m), jnp.exp(w2 - m), jnp.exp(w3 - m)
    emb = (x1 * e1 + x2 * e2 + x3 * xc) / (x1 + x2 + x3)
    emb = emb + jnp.dot(emb, rW_ref[...], preferred_element_type=jnp.float32) + rb_ref[...]
    h = jnp.tanh(jnp.dot(emb, mW1_ref[...], preferred_element_type=jnp.float32) + mb1_ref[...])
    logits = jnp.dot(h, mW2_ref[...], preferred_element_type=jnp.float32) + mb2_ref[...]
    mx = jnp.max(logits, axis=1, keepdims=True)
    lse = jnp.log(jnp.sum(jnp.exp(logits - mx), axis=1, keepdims=True)) + mx
    out_ref[...] = logits - lse


def _tcc(po1, po2, q2s, q2f, p):
    full = lambda shape: pl.BlockSpec(shape, lambda i: (0,) * len(shape))
    blk = lambda w: pl.BlockSpec((_R, w), lambda i: (i, 0))
    pblk = pl.BlockSpec((NC, _R, D), lambda i: (0, i, 0))
    return pl.pallas_call(
        _tcc_body,
        grid=(N // _R,),
        in_specs=[pblk] * 4 +
                 [full((1, 64)),
                  full((64, 16)), full((1, 16)), full((1, 16)),
                  full((64, 64)), full((1, 64)),
                  full((64, 16)), full((1, 16)), full((16, C)), full((1, C))],
        out_specs=blk(C),
        out_shape=jax.ShapeDtypeStruct((N, C), jnp.float32),
    )(po1, po2, q2s, q2f,
      p["gcn"]["b2"].reshape(1, 64),
      p["att"]["W1"], p["att"]["b1"].reshape(1, 16), p["att"]["W2"].reshape(1, 16),
      p["res"]["W"], p["res"]["b"].reshape(1, 64),
      p["mlp"]["W1"], p["mlp"]["b1"].reshape(1, 16),
      p["mlp"]["W2"], p["mlp"]["b2"].reshape(1, C))


def kernel(x, sadj, fadj, asadj, afadj, params):
    sadj, fadj = sadj.reshape(2 * E), fadj.reshape(2 * E)
    asadj, afadj = asadj.reshape(2 * E), afadj.reshape(2 * E)
    pa1, pb1, pa2, pb2, *rest = _tca(x, params)
    ss1, sd1, ss2, sd2 = (o.T for o in rest[0:4])
    g1 = rest[4]

    wpass = _sc_wpass()
    pairp = _sc_pair()
    w_pass = _sc_pass(1)
    plain_pass = _sc_pass(0)

    W1a, dd1a = wpass(ss1[0], sd1[0], ss1[1], sd1[1], asadj)
    W1b, dd1b = wpass(ss1[2], sd1[2], ss1[3], sd1[3], asadj)
    W2a, dd2a = wpass(ss2[0], sd2[0], ss2[1], sd2[1], afadj)
    W2b, dd2b = wpass(ss2[2], sd2[2], ss2[3], sd2[3], afadj)

    ps = [pairp(pa1, W1a, asadj), pairp(pb1, W1b, asadj),
          pairp(pa2, W2a, afadj), pairp(pb2, W2b, afadj)]
    q1s = plain_pass(g1, sadj)
    q1f = plain_pass(g1, fadj)

    def dmat(dd):
        return (dd[0] + dd[1]).reshape(DR * 128)[:2 * N].reshape(N, 2)

    d1 = jnp.concatenate([dmat(dd1a), dmat(dd1b)], axis=1)
    d2 = jnp.concatenate([dmat(dd2a), dmat(dd2b)], axis=1)

    ho1, so1s, so1d, ho2, so2s, so2d, g2s, g2f = _tcb(ps, d1, d2, q1s, q1f, params)

    po1 = w_pass(ho1, so1s.reshape(N), so1d.reshape(N), asadj)
    po2 = w_pass(ho2, so2s.reshape(N), so2d.reshape(N), afadj)
    q2s = plain_pass(g2s, sadj)
    q2f = plain_pass(g2f, fadj)

    return _tcc(po1, po2, q2s, q2f, params)


# speedup vs baseline: 29.6947x; 1.0205x over previous
import functools

import jax
import jax.numpy as jnp
from jax import lax
from jax.experimental import pallas as pl
from jax.experimental.pallas import tpu as pltpu
from jax.experimental.pallas import tpu_sc as plsc

N = 10000
E = 320000
F = 128
C = 16

NC = 2
NS = 16
NW = NC * NS
EPT = E // NW
B = 80
NB = EPT // B
D = 128
SUB_ROWS = 640
SUB_CHUNKS = 8

_R = 1000


def _copy_range(sid, src_at, dst_at):
    for j in range(SUB_CHUNKS):
        start = pl.multiple_of(sid * SUB_ROWS + j * B, 8)

        @pl.when(start < N)
        def _():
            pltpu.sync_copy(src_at(start), dst_at(start))


MEGA = 5
NM = NB // MEGA
DR = 160


@functools.lru_cache(maxsize=None)
def _sc_wpass():
    mesh = plsc.VectorSubcoreMesh(core_axis_name="c", subcore_axis_name="s",
                                  num_cores=NC, num_subcores=NS)
    scratch = [
        pltpu.VMEM((MEGA * B,), jnp.int32),
        pltpu.VMEM((MEGA * B,), jnp.int32),
        [pltpu.VMEM((B,), jnp.float32) for _ in range(2)],
        [pltpu.VMEM((B,), jnp.float32) for _ in range(2)],
        pltpu.VMEM((DR, 128), jnp.float32),
        pltpu.VMEM((2, B), jnp.int32),
        pltpu.VMEM_SHARED((DR, 128), jnp.float32),
        [pltpu.SemaphoreType.DMA for _ in range(6)],
        pltpu.VMEM((N,), jnp.float32),
        pltpu.VMEM((N,), jnp.float32),
        pltpu.VMEM((N,), jnp.float32),
        pltpu.VMEM((N,), jnp.float32),
    ]

    def body(ssrc0_hbm, sdst0_hbm, ssrc1_hbm, sdst1_hbm, adj_hbm,
             w_hbm, den_hbm,
             src_m, dst_m, wb0s, wb1s, den_v, rid_v, denacc, sems,
             s0s, s0d, s1s, s1d):
        cid = lax.axis_index("c")
        sid = lax.axis_index("s")
        wid = cid * NS + sid

        def zden(i, carry):
            for j in range(8):
                den_v[i, pl.ds(j * 16, 16)] = jnp.zeros((16,), jnp.float32)
            return carry
        lax.fori_loop(0, DR, zden, 0)
        for r in range(2):
            for j in range(B // 16):
                rid_v[r, pl.ds(j * 16, 16)] = (
                    lax.iota(jnp.int32, 16) + (r * B + j * 16))
        pltpu.sync_copy(ssrc0_hbm, s0s)
        pltpu.sync_copy(sdst0_hbm, s0d)
        pltpu.sync_copy(ssrc1_hbm, s1s)
        pltpu.sync_copy(sdst1_hbm, s1d)

        @pl.when(sid == 0)
        def _():
            pltpu.sync_copy(den_v, denacc)
        plsc.subcore_barrier()

        def wcalc(idxs, idxd, ss, sd):
            ev = plsc.load_gather(ss, [idxs]) + plsc.load_gather(sd, [idxd])
            ev = jnp.where(ev >= 0.0, ev, 0.2 * ev)
            return jnp.exp(ev)

        def midx(m, off):
            return adj_hbm.at[pl.ds(off + wid * EPT + m * (MEGA * B), MEGA * B)]

        def missue(m):
            pltpu.async_copy(midx(m, 0), src_m, sems[2])
            pltpu.async_copy(midx(m, E), dst_m, sems[3])

        missue(0)

        def mega(m, carry):
            mbase = wid * EPT + m * (MEGA * B)
            pltpu.make_async_copy(midx(m, 0), src_m, sems[2]).wait()
            pltpu.make_async_copy(midx(m, E), dst_m, sems[3]).wait()
            for j in range(MEGA):
                jb = j % 2
                wb0, wb1 = wb0s[jb], wb1s[jb]
                base = mbase + j * B

                @pl.when(m * MEGA + j >= 2)
                def _():
                    pltpu.make_async_copy(
                        wb0, w_hbm.at[pl.ds(base, B)], sems[jb]).wait()
                    pltpu.make_async_copy(
                        wb1, w_hbm.at[pl.ds(E + base, B)], sems[jb]).wait()
                for e0 in range(0, B, 16):
                    idxs = src_m[pl.ds(j * B + e0, 16)]
                    idxd = dst_m[pl.ds(j * B + e0, 16)]
                    flat = idxd * 2
                    w0 = wcalc(idxs, idxd, s0s, s0d)
                    wb0[pl.ds(e0, 16)] = w0
                    plsc.addupdate_scatter(
                        den_v, [lax.shift_right_logical(flat, 7),
                                lax.bitwise_and(flat, 127)], w0)
                    w1 = wcalc(idxs, idxd, s1s, s1d)
                    wb1[pl.ds(e0, 16)] = w1
                    plsc.addupdate_scatter(
                        den_v, [lax.shift_right_logical(flat + 1, 7),
                                lax.bitwise_and(flat + 1, 127)], w1)
                pltpu.async_copy(wb0, w_hbm.at[pl.ds(base, B)], sems[jb])
                pltpu.async_copy(wb1, w_hbm.at[pl.ds(E + base, B)], sems[jb])

            @pl.when(m + 1 < NM)
            def _():
                missue(m + 1)
            return carry
        lax.fori_loop(0, NM, mega, 0)
        for jb in range(2):
            pltpu.make_async_copy(wb0s[jb], w_hbm.at[pl.ds(0, B)], sems[jb]).wait()
            pltpu.make_async_copy(wb1s[jb], w_hbm.at[pl.ds(0, B)], sems[jb]).wait()

        pltpu.sync_copy(den_v.at[pl.ds(0, B)], denacc.at[rid_v.at[0]], add=True)
        pltpu.sync_copy(den_v.at[pl.ds(B, B)], denacc.at[rid_v.at[1]], add=True)
        plsc.subcore_barrier()

        @pl.when(sid == 0)
        def _():
            pltpu.sync_copy(denacc, den_hbm.at[cid])

    cp = pltpu.CompilerParams(needs_layout_passes=False)
    return pl.kernel(
        body,
        out_type=(jax.ShapeDtypeStruct((2 * E,), jnp.float32),
                  jax.ShapeDtypeStruct((NC, DR, 128), jnp.float32)),
        mesh=mesh, scratch_types=scratch, compiler_params=cp)


@functools.lru_cache(maxsize=None)
def _sc_pair():
    mesh = plsc.VectorSubcoreMesh(core_axis_name="c", subcore_axis_name="s",
                                  num_cores=NC, num_subcores=NS)
    scratch = [
        pltpu.VMEM((MEGA * B,), jnp.int32),
        [pltpu.VMEM((B,), jnp.int32) for _ in range(2)],
        [pltpu.VMEM((B, D), jnp.float32) for _ in range(2)],
        [pltpu.VMEM((B,), jnp.float32) for _ in range(2)],
        [pltpu.VMEM((B,), jnp.float32) for _ in range(2)],
        pltpu.VMEM_SHARED((N, D), jnp.float32),
        [pltpu.SemaphoreType.DMA for _ in range(7)],
    ]

    def body(rows_hbm, w_hbm, adj_hbm, out_hbm,
             src_m, dst_vs, rows_vs, wb0s, wb1s, acc, sems):
        cid = lax.axis_index("c")
        sid = lax.axis_index("s")
        wid = cid * NS + sid
        rows_v = rows_vs[0]

        def zrow(i, carry):
            for j in range(D // 16):
                rows_v[i, pl.ds(j * 16, 16)] = jnp.zeros((16,), jnp.float32)
            return carry
        lax.fori_loop(0, B, zrow, 0)
        _copy_range(sid, lambda s: rows_v, lambda s: acc.at[pl.ds(s, B)])
        plsc.subcore_barrier()

        def midx(m):
            return adj_hbm.at[pl.ds(wid * EPT + m * (MEGA * B), MEGA * B)]

        def issue(mbase, j):
            jb = j % 2
            pltpu.async_copy(adj_hbm.at[pl.ds(E + mbase + j * B, B)],
                             dst_vs[jb], sems[jb])
            pltpu.async_copy(rows_hbm.at[src_m.at[pl.ds(j * B, B)]],
                             rows_vs[jb], sems[2 + jb])
            pltpu.async_copy(w_hbm.at[pl.ds(mbase + j * B, B)],
                             wb0s[jb], sems[4 + jb])
            pltpu.async_copy(w_hbm.at[pl.ds(E + mbase + j * B, B)],
                             wb1s[jb], sems[4 + jb])

        pltpu.async_copy(midx(0), src_m, sems[6])

        def mega(m, carry):
            mbase = wid * EPT + m * (MEGA * B)
            pltpu.make_async_copy(midx(m), src_m, sems[6]).wait()
            issue(mbase, 0)
            for j in range(MEGA):
                jb = j % 2
                dst_v, rows_v = dst_vs[jb], rows_vs[jb]
                wb0, wb1 = wb0s[jb], wb1s[jb]
                pltpu.make_async_copy(adj_hbm.at[pl.ds(E + mbase + j * B, B)],
                                      dst_v, sems[jb]).wait()
                pltpu.make_async_copy(rows_hbm.at[src_m.at[pl.ds(j * B, B)]],
                                      rows_v, sems[2 + jb]).wait()
                pltpu.make_async_copy(w_hbm.at[pl.ds(mbase + j * B, B)],
                                      wb0, sems[4 + jb]).wait()
                pltpu.make_async_copy(w_hbm.at[pl.ds(E + mbase + j * B, B)],
                                      wb1, sems[4 + jb]).wait()
                if j + 1 < MEGA:
                    issue(mbase, j + 1)

                def scale(i, carry):
                    for u in range(2):
                        e = i * 2 + u
                        ee = jnp.full((16,), e, jnp.int32)
                        w0 = plsc.load_gather(wb0, [ee])
                        w1 = plsc.load_gather(wb1, [ee])
                        for jj in range(4):
                            sl = pl.ds(jj * 16, 16)
                            rows_v[e, sl] = rows_v[e, sl] * w0
                        for jj in range(4, 8):
                            sl = pl.ds(jj * 16, 16)
                            rows_v[e, sl] = rows_v[e, sl] * w1
                    return carry
                lax.fori_loop(0, B // 2, scale, 0)
                pltpu.sync_copy(rows_v, acc.at[dst_v], add=True)

            @pl.when(m + 1 < NM)
            def _():
                pltpu.async_copy(midx(m + 1), src_m, sems[6])
            return carry
        lax.fori_loop(0, NM, mega, 0)
        plsc.subcore_barrier()

        _copy_range(sid, lambda s: acc.at[pl.ds(s, B)],
                    lambda s: out_hbm.at[cid, pl.ds(s, B)])

    cp = pltpu.CompilerParams(needs_layout_passes=False)
    return pl.kernel(
        body,
        out_type=jax.ShapeDtypeStruct((NC, N, D), jnp.float32),
        mesh=mesh, scratch_types=scratch, compiler_params=cp)


@functools.lru_cache(maxsize=None)
def _sc_pass(H):
    mesh = plsc.VectorSubcoreMesh(core_axis_name="c", subcore_axis_name="s",
                                  num_cores=NC, num_subcores=NS)
    scratch = [
        pltpu.VMEM((MEGA * B,), jnp.int32),
        [pltpu.VMEM((B,), jnp.int32) for _ in range(2)],
        [pltpu.VMEM((B, D), jnp.float32) for _ in range(2)],
        pltpu.VMEM_SHARED((N, D), jnp.float32),
        [pltpu.SemaphoreType.DMA for _ in range(5)],
    ]
    if H:
        scratch += [
            pltpu.VMEM((N,), jnp.float32),
            pltpu.VMEM((N,), jnp.float32),
            pltpu.VMEM((B,), jnp.float32),
        ]

    def body(rows_hbm, ssrc_hbm, sdst_hbm, adj_hbm, out_hbm,
             src_m, dst_vs, rows_vs, acc, sems, *wscratch):
        cid = lax.axis_index("c")
        sid = lax.axis_index("s")
        wid = cid * NS + sid
        rows_v = rows_vs[0]

        def zrow(i, carry):
            for j in range(D // 16):
                rows_v[i, pl.ds(j * 16, 16)] = jnp.zeros((16,), jnp.float32)
            return carry
        lax.fori_loop(0, B, zrow, 0)
        _copy_range(sid, lambda s: rows_v, lambda s: acc.at[pl.ds(s, B)])

        if H:
            s_src_v, s_dst_v, wbuf = wscratch
            pltpu.sync_copy(ssrc_hbm, s_src_v)
            pltpu.sync_copy(sdst_hbm, s_dst_v)
        plsc.subcore_barrier()

        def midx(m):
            return adj_hbm.at[pl.ds(wid * EPT + m * (MEGA * B), MEGA * B)]

        def issue(mbase, j):
            jb = j % 2
            pltpu.async_copy(adj_hbm.at[pl.ds(E + mbase + j * B, B)],
                             dst_vs[jb], sems[jb])
            pltpu.async_copy(rows_hbm.at[src_m.at[pl.ds(j * B, B)]],
                             rows_vs[jb], sems[2 + jb])

        pltpu.async_copy(midx(0), src_m, sems[4])

        def mega(m, carry):
            mbase = wid * EPT + m * (MEGA * B)
            pltpu.make_async_copy(midx(m), src_m, sems[4]).wait()
            issue(mbase, 0)
            for j in range(MEGA):
                jb = j % 2
                dst_v, rows_v = dst_vs[jb], rows_vs[jb]
                pltpu.make_async_copy(adj_hbm.at[pl.ds(E + mbase + j * B, B)],
                                      dst_v, sems[jb]).wait()
                pltpu.make_async_copy(rows_hbm.at[src_m.at[pl.ds(j * B, B)]],
                                      rows_v, sems[2 + jb]).wait()
                if j + 1 < MEGA:
                    issue(mbase, j + 1)
                if H:
                    for e0 in range(0, B, 16):
                        idxs = src_m[pl.ds(j * B + e0, 16)]
                        idxd = dst_v[pl.ds(e0, 16)]
                        ev = (plsc.load_gather(s_src_v, [idxs])
                              + plsc.load_gather(s_dst_v, [idxd]))
                        ev = jnp.where(ev >= 0.0, ev, 0.2 * ev)
                        wbuf[pl.ds(e0, 16)] = jnp.exp(ev)

                    def scale(i, carry):
                        for u in range(4):
                            e = i * 4 + u
                            w = plsc.load_gather(wbuf, [jnp.full((16,), e, jnp.int32)])
                            for jj in range(5):
                                sl = pl.ds(jj * 16, 16)
                                rows_v[e, sl] = rows_v[e, sl] * w
                        return carry
                    lax.fori_loop(0, B // 4, scale, 0)
                pltpu.sync_copy(rows_v, acc.at[dst_v], add=True)

            @pl.when(m + 1 < NM)
            def _():
                pltpu.async_copy(midx(m + 1), src_m, sems[4])
            return carry
        lax.fori_loop(0, NM, mega, 0)
        plsc.subcore_barrier()

        _copy_range(sid, lambda s: acc.at[pl.ds(s, B)],
                    lambda s: out_hbm.at[cid, pl.ds(s, B)])

    cp = pltpu.CompilerParams(needs_layout_passes=False)
    if H:
        return pl.kernel(body,
                         out_type=jax.ShapeDtypeStruct((NC, N, D), jnp.float32),
                         mesh=mesh, scratch_types=scratch, compiler_params=cp)
    def body0(rows_hbm, adj_hbm, out_hbm, *rest):
        return body(rows_hbm, None, None, adj_hbm, out_hbm, *rest)
    return pl.kernel(body0,
                     out_type=jax.ShapeDtypeStruct((NC, N, D), jnp.float32),
                     mesh=mesh, scratch_types=scratch, compiler_params=cp)


def _ones_pad(ref, r):
    lane = lax.broadcasted_iota(jnp.int32, (r, 16), 1)
    ref[:, 64:80] = (lane < 1).astype(jnp.float32)
    ref[:, 80:128] = jnp.zeros((r, 48), jnp.float32)


def _tca_body(x_ref, *refs):
    (w10, w11, w12, w13, w20, w21, w22, w23, as1_ref, ad1_ref, as2_ref, ad2_ref,
     wg_ref,
     pa1, pb1, pa2, pb2,
     ss1_ref, sd1_ref, ss2_ref, sd2_ref, g1_ref) = refs
    x = x_ref[...]
    for (ws, prefs, ss_ref, sd_ref, as_ref, ad_ref) in [
            ((w10, w11, w12, w13), (pa1, pb1), ss1_ref, sd1_ref, as1_ref, ad1_ref),
            ((w20, w21, w22, w23), (pa2, pb2), ss2_ref, sd2_ref, as2_ref, ad2_ref)]:
        for k in range(4):
            h = jnp.dot(x, ws[k][...], preferred_element_type=jnp.float32)
            c = (k % 2) * 64
            prefs[k // 2][:, c:c + 64] = h
            ss_ref[:, k:k + 1] = jnp.sum(h * as_ref[k:k + 1, :], axis=1, keepdims=True)
            sd_ref[:, k:k + 1] = jnp.sum(h * ad_ref[k:k + 1, :], axis=1, keepdims=True)
    g1_ref[:, 0:64] = jnp.dot(x, wg_ref[...], preferred_element_type=jnp.float32)
    g1_ref[:, 64:128] = jnp.zeros((_R, 64), jnp.float32)


def _tca(x, p):
    full = lambda shape: pl.BlockSpec(shape, lambda i: (0,) * len(shape))
    blk = lambda w: pl.BlockSpec((_R, w), lambda i: (i, 0))
    sblk = pl.BlockSpec((_R, 4), lambda i: (i, 0))
    g1h, g2h = p["gat1"]["heads"], p["gat2"]["heads"]
    asrc1 = jnp.stack([h["a_src"] for h in g1h])
    adst1 = jnp.stack([h["a_dst"] for h in g1h])
    asrc2 = jnp.stack([h["a_src"] for h in g2h])
    adst2 = jnp.stack([h["a_dst"] for h in g2h])
    outs = [jax.ShapeDtypeStruct((N, D), jnp.float32)] * 4 + \
           [jax.ShapeDtypeStruct((N, 4), jnp.float32)] * 4 + \
           [jax.ShapeDtypeStruct((N, D), jnp.float32)]
    return pl.pallas_call(
        _tca_body,
        grid=(N // _R,),
        in_specs=[blk(F)] + [full((F, 64))] * 8 + [full((4, 64))] * 4 + [full((F, 64))],
        out_specs=[blk(D)] * 4 + [sblk] * 4 + [blk(D)],
        out_shape=outs,
    )(x, *[h["W"] for h in g1h], *[h["W"] for h in g2h],
      asrc1, adst1, asrc2, adst2, p["gcn"]["W1"])


def _norm(pref):
    acc = pref[0] + pref[1]
    return acc[:, 0:64] / (acc[:, 64:65] + 1e-16)


def _tcb_body(p1a, p1b, p2a, p2b, d1_ref, d2_ref, q1s_ref, q1f_ref,
              wo1_ref, as1_ref, ad1_ref, wo2_ref, as2_ref, ad2_ref,
              b1_ref, w2_ref,
              ho1_ref, so1s_ref, so1d_ref, ho2_ref, so2s_ref, so2d_ref,
              g2s_ref, g2f_ref):
    for (pa, pb, d_ref, wo_ref, as_ref, ad_ref, ho_ref, sos_ref, sod_ref) in [
            (p1a, p1b, d1_ref, wo1_ref, as1_ref, ad1_ref, ho1_ref, so1s_ref, so1d_ref),
            (p2a, p2b, d2_ref, wo2_ref, as2_ref, ad2_ref, ho2_ref, so2s_ref, so2d_ref)]:
        acca = pa[0] + pa[1]
        accb = pb[0] + pb[1]
        hcat = jnp.concatenate(
            [acca[:, 0:64] / (d_ref[:, 0:1] + 1e-16),
             acca[:, 64:128] / (d_ref[:, 1:2] + 1e-16),
             accb[:, 0:64] / (d_ref[:, 2:3] + 1e-16),
             accb[:, 64:128] / (d_ref[:, 3:4] + 1e-16)], axis=1)
        ho = jnp.dot(hcat, wo_ref[...], preferred_element_type=jnp.float32)
        ho_ref[:, 0:64] = ho
        _ones_pad(ho_ref, _R)
        sos_ref[...] = jnp.sum(ho * as_ref[...], axis=1, keepdims=True)
        sod_ref[...] = jnp.sum(ho * ad_ref[...], axis=1, keepdims=True)
    b1 = b1_ref[...]
    w2 = w2_ref[...]
    zero = jnp.zeros((_R, 64), jnp.float32)
    hs = jax.nn.relu(q1s_ref[0, :, 0:64] + q1s_ref[1, :, 0:64] + b1)
    hf = jax.nn.relu(q1f_ref[0, :, 0:64] + q1f_ref[1, :, 0:64] + b1)
    g2s_ref[:, 0:64] = jnp.dot(hs, w2, preferred_element_type=jnp.float32)
    g2s_ref[:, 64:128] = zero
    g2f_ref[:, 0:64] = jnp.dot(hf, w2, preferred_element_type=jnp.float32)
    g2f_ref[:, 64:128] = zero


def _tcb(ps, d1, d2, q1s, q1f, p):
    full = lambda shape: pl.BlockSpec(shape, lambda i: (0,) * len(shape))
    blk = lambda w: pl.BlockSpec((_R, w), lambda i: (i, 0))
    pblk = pl.BlockSpec((NC, _R, D), lambda i: (0, i, 0))
    sblk = pl.BlockSpec((_R, 1), lambda i: (i, 0))
    dblk = pl.BlockSpec((_R, 4), lambda i: (i, 0))
    outs = [jax.ShapeDtypeStruct((N, D), jnp.float32),
            jax.ShapeDtypeStruct((N, 1), jnp.float32),
            jax.ShapeDtypeStruct((N, 1), jnp.float32)] * 2 + \
           [jax.ShapeDtypeStruct((N, D), jnp.float32)] * 2
    o1, o2 = p["gat1"]["out"], p["gat2"]["out"]
    return pl.pallas_call(
        _tcb_body,
        grid=(N // _R,),
        in_specs=[pblk] * 4 + [dblk] * 2 + [pblk] * 2 +
                 [full((256, 64)), full((1, 64)), full((1, 64))] * 2 +
                 [full((1, 64)), full((64, 64))],
        out_specs=[blk(D), sblk, sblk, blk(D), sblk, sblk, blk(D), blk(D)],
        out_shape=outs,
    )(*ps, d1, d2, q1s, q1f,
      o1["W"], o1["a_src"].reshape(1, 64), o1["a_dst"].reshape(1, 64),
      o2["W"], o2["a_src"].reshape(1, 64), o2["a_dst"].reshape(1, 64),
      p["gcn"]["b1"].reshape(1, 64), p["gcn"]["W2"])


def _tcc_body(po1_ref, po2_ref, q2s_ref, q2f_ref,
              b2_ref, aW1_ref, ab1_ref, aW2_ref, rW_ref, rb_ref,
              mW1_ref, mb1_ref, mW2_ref, mb2_ref, out_ref):
    def emb_of(po_ref):
        h = _norm(po_ref)
        h = jnp.where(h > 0.0, h, jnp.exp(jnp.minimum(h, 0.0)) - 1.0)
        m = jnp.max(h, axis=1, keepdims=True)
        ex = jnp.exp(h - m)
        return ex / jnp.sum(ex, axis=1, keepdims=True)

    e1 = emb_of(po1_ref)
    e2 = emb_of(po2_ref)
    b2 = b2_ref[...]
    com1 = jax.nn.relu(q2s_ref[0, :, 0:64] + q2s_ref[1, :, 0:64] + b2)
    com2 = jax.nn.relu(q2f_ref[0, :, 0:64] + q2f_ref[1, :, 0:64] + b2)
    xc = (com1 + com2) * 0.5

    aW1, ab1, aW2 = aW1_ref[...], ab1_ref[...], aW2_ref[...]

    def att_w(e):
        t = jnp.tanh(jnp.dot(e, aW1, preferred_element_type=jnp.float32) + ab1)
        return jnp.sum(t * aW2, axis=1, keepdims=True)

    w1, w2, w3 = att_w(e1), att_w(e2), att_w(xc)
    m = jnp.maximum(jnp.maximum(w1, w2), w3)
    x1, x2, x3 = jnp.exp(w1 - m), jnp.exp(w2 - m), jnp.exp(w3 - m)
    emb = (x1 * e1 + x2 * e2 + x3 * xc) / (x1 + x2 + x3)
    emb = emb + jnp.dot(emb, rW_ref[...], preferred_element_type=jnp.float32) + rb_ref[...]
    h = jnp.tanh(jnp.dot(emb, mW1_ref[...], preferred_element_type=jnp.float32) + mb1_ref[...])
    logits = jnp.dot(h, mW2_ref[...], preferred_element_type=jnp.float32) + mb2_ref[...]
    mx = jnp.max(logits, axis=1, keepdims=True)
    lse = jnp.log(jnp.sum(jnp.exp(logits - mx), axis=1, keepdims=True)) + mx
    out_ref[...] = logits - lse


def _tcc(po1, po2, q2s, q2f, p):
    full = lambda shape: pl.BlockSpec(shape, lambda i: (0,) * len(shape))
    blk = lambda w: pl.BlockSpec((_R, w), lambda i: (i, 0))
    pblk = pl.BlockSpec((NC, _R, D), lambda i: (0, i, 0))
    return pl.pallas_call(
        _tcc_body,
        grid=(N // _R,),
        in_specs=[pblk] * 4 +
                 [full((1, 64)),
                  full((64, 16)), full((1, 16)), full((1, 16)),
                  full((64, 64)), full((1, 64)),
                  full((64, 16)), full((1, 16)), full((16, C)), full((1, C))],
        out_specs=blk(C),
        out_shape=jax.ShapeDtypeStruct((N, C), jnp.float32),
    )(po1, po2, q2s, q2f,
      p["gcn"]["b2"].reshape(1, 64),
      p["att"]["W1"], p["att"]["b1"].reshape(1, 16), p["att"]["W2"].reshape(1, 16),
      p["res"]["W"], p["res"]["b"].reshape(1, 64),
      p["mlp"]["W1"], p["mlp"]["b1"].reshape(1, 16),
      p["mlp"]["W2"], p["mlp"]["b2"].reshape(1, C))


def kernel(x, sadj, fadj, asadj, afadj, params):
    sadj, fadj = sadj.reshape(2 * E), fadj.reshape(2 * E)
    asadj, afadj = asadj.reshape(2 * E), afadj.reshape(2 * E)
    pa1, pb1, pa2, pb2, *rest = _tca(x, params)
    ss1, sd1, ss2, sd2 = (o.T for o in rest[0:4])
    g1 = rest[4]

    wpass = _sc_wpass()
    pairp = _sc_pair()
    w_pass = _sc_pass(1)
    plain_pass = _sc_pass(0)

    W1a, dd1a = wpass(ss1[0], sd1[0], ss1[1], sd1[1], asadj)
    W1b, dd1b = wpass(ss1[2], sd1[2], ss1[3], sd1[3], asadj)
    W2a, dd2a = wpass(ss2[0], sd2[0], ss2[1], sd2[1], afadj)
    W2b, dd2b = wpass(ss2[2], sd2[2], ss2[3], sd2[3], afadj)

    ps = [pairp(pa1, W1a, asadj), pairp(pb1, W1b, asadj),
          pairp(pa2, W2a, afadj), pairp(pb2, W2b, afadj)]
    q1s = plain_pass(g1, sadj)
    q1f = plain_pass(g1, fadj)

    def dmat(dd):
        return (dd[0] + dd[1]).reshape(DR * 128)[:2 * N].reshape(N, 2)

    d1 = jnp.concatenate([dmat(dd1a), dmat(dd1b)], axis=1)
    d2 = jnp.concatenate([dmat(dd2a), dmat(dd2b)], axis=1)

    ho1, so1s, so1d, ho2, so2s, so2d, g2s, g2f = _tcb(ps, d1, d2, q1s, q1f, params)

    po1 = w_pass(ho1, so1s.reshape(N), so1d.reshape(N), asadj)
    po2 = w_pass(ho2, so2s.reshape(N), so2d.reshape(N), afadj)
    q2s = plain_pass(g2s, sadj)
    q2f = plain_pass(g2f, fadj)

    return _tcc(po1, po2, q2s, q2f, params)
